# Initial kernel scaffold; baseline (speedup 1.0000x reference)
import jax
import jax.numpy as jnp
from jax.experimental import pallas as pl


def kernel(label_ids, pos_ids, label_table, pos_table):
    raise NotImplementedError("write your pallas kernel here")


# speedup vs baseline: 4.0603x; 4.0603x over previous
import functools

import jax
import jax.numpy as jnp
from jax import lax
from jax.experimental import pallas as pl
from jax.experimental.pallas import tpu as pltpu
from jax.experimental.pallas import tpu_sc as plsc

PAD_ID = 0
D = 64
NC, NS = 2, 16
NW = NC * NS
CHUNK = 128


@functools.partial(jax.jit, static_argnames=("n",))
def _gather_both(lab_idx, pos_idx, lab_tab, pos_tab, n):
    bpw = n // NW
    nchunk = bpw // CHUNK
    mesh = plsc.VectorSubcoreMesh(core_axis_name="c", subcore_axis_name="s")

    @functools.partial(
        pl.kernel,
        mesh=mesh,
        compiler_params=pltpu.CompilerParams(use_tc_tiling_on_sc=False),
        out_type=(
            jax.ShapeDtypeStruct((n, D), jnp.float32),
            jax.ShapeDtypeStruct((n, D), jnp.float32),
        ),
        scratch_types=[
            pltpu.VMEM((bpw,), jnp.int32),
            pltpu.VMEM((bpw,), jnp.int32),
            pltpu.VMEM((CHUNK, D), jnp.float32),
            pltpu.VMEM((CHUNK, D), jnp.float32),
            pltpu.SemaphoreType.DMA,
            pltpu.SemaphoreType.DMA,
        ],
    )
    def body(lab_idx_hbm, pos_idx_hbm, lab_tab_hbm, pos_tab_hbm,
             pos_out_hbm, lab_out_hbm,
             lab_idx_v, pos_idx_v, lab_rows, pos_rows, sem_l, sem_p):
        wid = lax.axis_index("s") * NC + lax.axis_index("c")
        base = wid * bpw
        pltpu.sync_copy(lab_idx_hbm.at[pl.ds(base, bpw)], lab_idx_v)
        pltpu.sync_copy(pos_idx_hbm.at[pl.ds(base, bpw)], pos_idx_v)

        def step(j, carry):
            off = j * CHUNK
            cl = pltpu.async_copy(
                lab_tab_hbm.at[lab_idx_v.at[pl.ds(off, CHUNK)]], lab_rows, sem_l)
            cp = pltpu.async_copy(
                pos_tab_hbm.at[pos_idx_v.at[pl.ds(off, CHUNK)]], pos_rows, sem_p)
            cl.wait()
            pltpu.sync_copy(lab_rows, lab_out_hbm.at[pl.ds(base + off, CHUNK)])
            cp.wait()
            pltpu.sync_copy(pos_rows, pos_out_hbm.at[pl.ds(base + off, CHUNK)])
            return carry

        lax.fori_loop(0, nchunk, step, 0)

    return body(lab_idx, pos_idx, lab_tab, pos_tab)


def kernel(label_ids, pos_ids, label_table, pos_table):
    b, l = label_ids.shape
    n = b * l
    pos_table = pos_table.at[PAD_ID].set(0.0)
    lab_idx = label_ids.reshape(n).astype(jnp.int32)
    pos_idx = pos_ids.reshape(n).astype(jnp.int32)
    pos_out, lab_out = _gather_both(lab_idx, pos_idx, label_table, pos_table, n)
    return pos_out.reshape(b, l, D), lab_out.reshape(b, l, D)


# speedup vs baseline: 4.1493x; 1.0219x over previous
import functools

import jax
import jax.numpy as jnp
from jax import lax
from jax.experimental import pallas as pl
from jax.experimental.pallas import tpu as pltpu
from jax.experimental.pallas import tpu_sc as plsc

PAD_ID = 0
D = 64
NC, NS = 2, 16
NW = NC * NS
CHUNK = 128
NBUF = 4


@functools.partial(jax.jit, static_argnames=("n",))
def _gather_both(lab_idx, pos_idx, lab_tab, pos_tab, n):
    bpw = n // NW
    nchunk = bpw // CHUNK
    mesh = plsc.VectorSubcoreMesh(core_axis_name="c", subcore_axis_name="s")

    @functools.partial(
        pl.kernel,
        mesh=mesh,
        compiler_params=pltpu.CompilerParams(use_tc_tiling_on_sc=False),
        out_type=(
            jax.ShapeDtypeStruct((n, D), jnp.float32),
            jax.ShapeDtypeStruct((n, D), jnp.float32),
        ),
        scratch_types=[
            pltpu.VMEM((bpw,), jnp.int32),
            pltpu.VMEM((bpw,), jnp.int32),
            pltpu.VMEM((NBUF, CHUNK, D), jnp.float32),
            pltpu.VMEM((NBUF, CHUNK, D), jnp.float32),
            pltpu.SemaphoreType.DMA,
            pltpu.SemaphoreType.DMA,
            pltpu.SemaphoreType.DMA,
            pltpu.SemaphoreType.DMA,
        ],
    )
    def body(lab_idx_hbm, pos_idx_hbm, lab_tab_hbm, pos_tab_hbm,
             pos_out_hbm, lab_out_hbm,
             lab_idx_v, pos_idx_v, lab_rows, pos_rows,
             sem_gl, sem_gp, sem_wl, sem_wp):
        wid = lax.axis_index("s") * NC + lax.axis_index("c")
        base = wid * bpw
        pltpu.sync_copy(lab_idx_hbm.at[pl.ds(base, bpw)], lab_idx_v)
        pltpu.sync_copy(pos_idx_hbm.at[pl.ds(base, bpw)], pos_idx_v)

        def gather(j, b):
            return (
                pltpu.make_async_copy(
                    lab_tab_hbm.at[lab_idx_v.at[pl.ds(j * CHUNK, CHUNK)]],
                    lab_rows.at[b], sem_gl),
                pltpu.make_async_copy(
                    pos_tab_hbm.at[pos_idx_v.at[pl.ds(j * CHUNK, CHUNK)]],
                    pos_rows.at[b], sem_gp),
            )

        def writeback(j, b):
            return (
                pltpu.make_async_copy(
                    lab_rows.at[b],
                    lab_out_hbm.at[pl.ds(base + j * CHUNK, CHUNK)], sem_wl),
                pltpu.make_async_copy(
                    pos_rows.at[b],
                    pos_out_hbm.at[pl.ds(base + j * CHUNK, CHUNK)], sem_wp),
            )

        ngroups = nchunk // NBUF

        def group(g, carry):
            for b in range(NBUF):
                j = g * NBUF + b

                @pl.when(g >= 1)
                def _(b=b, j=j):
                    for c in writeback(j - NBUF, b):
                        c.wait()

                for c in gather(j, b):
                    c.start()
            for b in range(NBUF):
                j = g * NBUF + b
                for c in gather(j, b):
                    c.wait()
                for c in writeback(j, b):
                    c.start()
            return carry

        lax.fori_loop(0, ngroups, group, 0)
        for b in range(NBUF):
            for c in writeback((ngroups - 1) * NBUF + b, b):
                c.wait()

    return body(lab_idx, pos_idx, lab_tab, pos_tab)


def kernel(label_ids, pos_ids, label_table, pos_table):
    b, l = label_ids.shape
    n = b * l
    pos_table = pos_table.at[PAD_ID].set(0.0)
    lab_idx = label_ids.reshape(n).astype(jnp.int32)
    pos_idx = pos_ids.reshape(n).astype(jnp.int32)
    pos_out, lab_out = _gather_both(lab_idx, pos_idx, label_table, pos_table, n)
    return pos_out.reshape(b, l, D), lab_out.reshape(b, l, D)


# speedup vs baseline: 5.4103x; 1.3039x over previous
import functools

import jax
import jax.numpy as jnp
from jax import lax
from jax.experimental import pallas as pl
from jax.experimental.pallas import tpu as pltpu
from jax.experimental.pallas import tpu_sc as plsc

PAD_ID = 0
D = 64
DP = 128
NC, NS = 2, 16
NW = NC * NS
NBUF = 2
NIDX = 4


@functools.partial(jax.jit, static_argnames=("b", "l"))
def _gather_both(lab_idx, pos_idx, lab_tab_pad, pos_tab_pad, b, l):
    bpw = b // NW
    c0 = min(128, l)
    c1 = l - c0
    lpad = 2 * 128
    mesh = plsc.VectorSubcoreMesh(core_axis_name="c", subcore_axis_name="s")

    @functools.partial(
        pl.kernel,
        mesh=mesh,
        out_type=(
            jax.ShapeDtypeStruct((b * l, DP), jnp.float32),
            jax.ShapeDtypeStruct((b * l, DP), jnp.float32),
        ),
        scratch_types=[
            pltpu.VMEM((NIDX * lpad,), jnp.int32),
            pltpu.VMEM((NIDX * lpad,), jnp.int32),
            pltpu.VMEM((NBUF, l, DP), jnp.float32),
            pltpu.VMEM((NBUF, l, DP), jnp.float32),
            pltpu.SemaphoreType.DMA,
            pltpu.SemaphoreType.DMA,
            pltpu.SemaphoreType.DMA,
            pltpu.SemaphoreType.DMA,
            pltpu.SemaphoreType.DMA,
            pltpu.SemaphoreType.DMA,
        ],
    )
    def body(lab_idx_hbm, pos_idx_hbm, lab_tab_hbm, pos_tab_hbm,
             pos_out_hbm, lab_out_hbm,
             lab_idx_v, pos_idx_v, lab_rows, pos_rows,
             sem_il, sem_ip, sem_gl, sem_gp, sem_wl, sem_wp):
        wid = lax.axis_index("s") * NC + lax.axis_index("c")
        base = wid * bpw

        def idx_copy(i):
            q0 = pl.multiple_of(lax.rem(i, NIDX) * lpad, lpad)
            off = (base + i) * l
            return (
                pltpu.make_async_copy(
                    lab_idx_hbm.at[pl.ds(off, l)],
                    lab_idx_v.at[pl.ds(q0, l)], sem_il),
                pltpu.make_async_copy(
                    pos_idx_hbm.at[pl.ds(off, l)],
                    pos_idx_v.at[pl.ds(q0, l)], sem_ip),
            )

        def gather(i):
            q0 = pl.multiple_of(lax.rem(i, NIDX) * lpad, lpad)
            r = lax.rem(i, NBUF)
            cs = (
                pltpu.make_async_copy(
                    lab_tab_hbm.at[lab_idx_v.at[pl.ds(q0, c0)]],
                    lab_rows.at[r, pl.ds(0, c0)], sem_gl),
                pltpu.make_async_copy(
                    pos_tab_hbm.at[pos_idx_v.at[pl.ds(q0, c0)]],
                    pos_rows.at[r, pl.ds(0, c0)], sem_gp),
            )
            if c1:
                cs += (
                    pltpu.make_async_copy(
                        lab_tab_hbm.at[lab_idx_v.at[pl.ds(q0 + c0, c1)]],
                        lab_rows.at[r, pl.ds(c0, c1)], sem_gl),
                    pltpu.make_async_copy(
                        pos_tab_hbm.at[pos_idx_v.at[pl.ds(q0 + c0, c1)]],
                        pos_rows.at[r, pl.ds(c0, c1)], sem_gp),
                )
            return cs

        def writeback(i):
            r = lax.rem(i, NBUF)
            off = (base + i) * l
            return (
                pltpu.make_async_copy(
                    lab_rows.at[r], lab_out_hbm.at[pl.ds(off, l)], sem_wl),
                pltpu.make_async_copy(
                    pos_rows.at[r], pos_out_hbm.at[pl.ds(off, l)], sem_wp),
            )

        for i in range(NBUF):
            for c in idx_copy(i):
                c.start()

        def step(i, carry):
            for c in idx_copy(i):
                c.wait()

            @pl.when(i >= NBUF)
            def _():
                for c in writeback(i - NBUF):
                    c.wait()

            for c in gather(i):
                c.start()

            @pl.when(i + NBUF < bpw)
            def _():
                for c in idx_copy(i + NBUF):
                    c.start()

            @pl.when(i >= 1)
            def _():
                for c in gather(i - 1):
                    c.wait()
                for c in writeback(i - 1):
                    c.start()

            return carry

        lax.fori_loop(0, bpw, step, 0)
        for c in gather(bpw - 1):
            c.wait()
        for c in writeback(bpw - 1):
            c.start()
        for i in range(NBUF):
            for c in writeback(bpw - NBUF + i):
                c.wait()

    return body(lab_idx, pos_idx, lab_tab_pad, pos_tab_pad)


def kernel(label_ids, pos_ids, label_table, pos_table):
    b, l = label_ids.shape
    pos_table = pos_table.at[PAD_ID].set(0.0)
    lab_tab_pad = jnp.pad(label_table, ((0, 0), (0, DP - D)))
    pos_tab_pad = jnp.pad(pos_table, ((0, 0), (0, DP - D)))
    lab_idx = label_ids.reshape(-1).astype(jnp.int32)
    pos_idx = pos_ids.reshape(-1).astype(jnp.int32)
    pos_out, lab_out = _gather_both(
        lab_idx, pos_idx, lab_tab_pad, pos_tab_pad, b, l)
    return (pos_out[:, :D].reshape(b, l, D),
            lab_out[:, :D].reshape(b, l, D))
